# Initial kernel scaffold; baseline (speedup 1.0000x reference)
import jax
import jax.numpy as jnp
from jax.experimental import pallas as pl


def kernel(x, router_W, A, Bw):
    raise NotImplementedError("write your pallas kernel here")


# speedup vs baseline: 10.8693x; 10.8693x over previous
import functools

import jax
import jax.numpy as jnp
from jax.experimental import pallas as pl

_E = 8
_R = 8
_TL = 512


def _moe_body(x_ref, cls_ref, rw_ref, aall_ref, ball_ref, out_ref):
    rw = rw_ref[...]
    b = pl.program_id(0)
    cls_b = cls_ref[pl.ds(b, 1), :]
    logits = jax.lax.dot_general(
        cls_b, rw, (((1,), (1,)), ((), ())),
        preferred_element_type=jnp.float32)
    iota = jax.lax.broadcasted_iota(jnp.int32, (1, _E), 1)
    big = jnp.int32(2 * _E)
    m1 = jnp.max(logits, axis=1, keepdims=True)
    i1 = jnp.min(jnp.where(logits == m1, iota, big), axis=1, keepdims=True)
    rest = jnp.where(iota == i1, jnp.float32(-1e30), logits)
    m2 = jnp.max(rest, axis=1, keepdims=True)
    i2 = jnp.min(jnp.where(rest == m2, iota, big), axis=1, keepdims=True)
    in_top = (iota == i1) | (iota == i2)
    ex = jnp.where(in_top, jnp.exp(logits - m1), jnp.float32(0.0))
    gates = ex / jnp.sum(ex, axis=1, keepdims=True)

    ii = jax.lax.broadcasted_iota(jnp.int32, (_E, _E * _R), 0)
    jj = jax.lax.broadcasted_iota(jnp.int32, (_E, _E * _R), 1)
    expand = jnp.where(jj // _R == ii, jnp.float32(1.0), jnp.float32(0.0))
    gates_rep = jnp.dot(gates, expand,
                        preferred_element_type=jnp.float32)

    xt = x_ref[0]
    down = jnp.dot(xt, aall_ref[...],
                   preferred_element_type=jnp.float32)
    down = down * gates_rep
    up = jnp.dot(down, ball_ref[...],
                 preferred_element_type=jnp.float32)
    out_ref[0] = xt + up


@jax.jit
def kernel(x, router_W, A, Bw):
    B, L, H = x.shape
    E, _, R = A.shape
    cls = x[:, 0, :]
    A_all = A.transpose(1, 0, 2).reshape(H, E * R)
    B_all = Bw.reshape(E * R, H)

    grid = (B, L // _TL)
    return pl.pallas_call(
        _moe_body,
        grid=grid,
        in_specs=[
            pl.BlockSpec((1, _TL, H), lambda b, l: (b, l, 0)),
            pl.BlockSpec((B, H), lambda b, l: (0, 0)),
            pl.BlockSpec((E, H), lambda b, l: (0, 0)),
            pl.BlockSpec((H, E * R), lambda b, l: (0, 0)),
            pl.BlockSpec((E * R, H), lambda b, l: (0, 0)),
        ],
        out_specs=pl.BlockSpec((1, _TL, H), lambda b, l: (b, l, 0)),
        out_shape=jax.ShapeDtypeStruct((B, L, H), x.dtype),
    )(x, cls, router_W, A_all, B_all)


# speedup vs baseline: 11.6038x; 1.0676x over previous
import functools

import jax
import jax.numpy as jnp
from jax.experimental import pallas as pl

_E = 8
_R = 8
_TL = 1024


def _moe_body(x_ref, cls_ref, rw_ref, aall_ref, ball_ref, out_ref):
    rw = rw_ref[...]
    b = pl.program_id(0)
    cls_b = cls_ref[pl.ds(b, 1), :]
    logits = jax.lax.dot_general(
        cls_b, rw, (((1,), (1,)), ((), ())),
        preferred_element_type=jnp.float32)
    iota = jax.lax.broadcasted_iota(jnp.int32, (1, _E), 1)
    big = jnp.int32(2 * _E)
    m1 = jnp.max(logits, axis=1, keepdims=True)
    i1 = jnp.min(jnp.where(logits == m1, iota, big), axis=1, keepdims=True)
    rest = jnp.where(iota == i1, jnp.float32(-1e30), logits)
    m2 = jnp.max(rest, axis=1, keepdims=True)
    i2 = jnp.min(jnp.where(rest == m2, iota, big), axis=1, keepdims=True)
    in_top = (iota == i1) | (iota == i2)
    ex = jnp.where(in_top, jnp.exp(logits - m1), jnp.float32(0.0))
    gates = ex / jnp.sum(ex, axis=1, keepdims=True)

    ii = jax.lax.broadcasted_iota(jnp.int32, (_E, _E * _R), 0)
    jj = jax.lax.broadcasted_iota(jnp.int32, (_E, _E * _R), 1)
    expand = jnp.where(jj // _R == ii, jnp.float32(1.0), jnp.float32(0.0))
    gates_rep = jnp.dot(gates, expand,
                        preferred_element_type=jnp.float32)

    xt = x_ref[0]
    down = jnp.dot(xt, aall_ref[...],
                   preferred_element_type=jnp.float32)
    down = down * gates_rep
    up = jnp.dot(down, ball_ref[...],
                 preferred_element_type=jnp.float32)
    out_ref[0] = xt + up


@jax.jit
def kernel(x, router_W, A, Bw):
    B, L, H = x.shape
    E, _, R = A.shape
    cls = x[:, 0, :]
    A_all = A.transpose(1, 0, 2).reshape(H, E * R)
    B_all = Bw.reshape(E * R, H)

    grid = (B, L // _TL)
    return pl.pallas_call(
        _moe_body,
        grid=grid,
        in_specs=[
            pl.BlockSpec((1, _TL, H), lambda b, l: (b, l, 0)),
            pl.BlockSpec((B, H), lambda b, l: (0, 0)),
            pl.BlockSpec((E, H), lambda b, l: (0, 0)),
            pl.BlockSpec((H, E * R), lambda b, l: (0, 0)),
            pl.BlockSpec((E * R, H), lambda b, l: (0, 0)),
        ],
        out_specs=pl.BlockSpec((1, _TL, H), lambda b, l: (b, l, 0)),
        out_shape=jax.ShapeDtypeStruct((B, L, H), x.dtype),
    )(x, cls, router_W, A_all, B_all)
